# baseline (speedup 1.0000x reference)
import functools

import jax
import jax.numpy as jnp
import numpy as np
from jax import lax
from jax.experimental import pallas as pl
from jax.experimental.pallas import tpu as pltpu
from jax.experimental.pallas import tpu_sc as plsc

_NUM_LEVELS = 16
_MIN_RES = 16
_MAX_RES = 1024
_TABLE_SIZE = 2 ** 19
_FEAT = 2
_N = 262144

_growth = np.exp((np.log(_MAX_RES) - np.log(_MIN_RES)) / (_NUM_LEVELS - 1))
_SCALES = [
    float(v)
    for v in np.floor(_MIN_RES * _growth ** np.arange(_NUM_LEVELS)).astype(np.float32)
]
_P1 = 2654435761
_P2 = 805459861
_MASK = _TABLE_SIZE - 1

_NW = 32
_C = 128
_PPW = _N // _NW
_CHUNKS = _PPW // _C
_R2 = _NUM_LEVELS * 8 * _FEAT


def _sc_encode(pT, table_flat):
    mesh = plsc.VectorSubcoreMesh(core_axis_name="c", subcore_axis_name="s")

    @functools.partial(
        pl.kernel,
        mesh=mesh,
        out_type=jax.ShapeDtypeStruct((_NUM_LEVELS * _FEAT, _N), jnp.float32),
        scratch_types=[
            pltpu.VMEM((3, _C), jnp.float32),
            pltpu.VMEM((_R2 * _C,), jnp.int32),
            pltpu.VMEM((_R2 * _C,), jnp.float32),
            pltpu.VMEM((_NUM_LEVELS * _FEAT, _C), jnp.float32),
            pltpu.SemaphoreType.DMA,
        ],
    )
    def enc_kernel(pT_hbm, table_hbm, enc_hbm, p_v, idx_v, rows_v, enc_v, gsem):
        wid = lax.axis_index("s") * 2 + lax.axis_index("c")

        def corners_1d(s):
            fi = s.astype(jnp.int32)
            ff = fi.astype(jnp.float32)
            ci = jnp.where(s == ff, fi, fi + 1)
            return fi.astype(jnp.uint32), ci.astype(jnp.uint32), s - ff

        def chunk_body(t, carry):
            base = (wid * _CHUNKS + t) * _C
            pltpu.sync_copy(pT_hbm.at[:, pl.ds(base, _C)], p_v)

            def hash_group(g, carry2):
                gb = g * 16
                x = p_v[0, pl.ds(gb, 16)]
                y = p_v[1, pl.ds(gb, 16)]
                z = p_v[2, pl.ds(gb, 16)]
                for l in range(_NUM_LEVELS):
                    sc = _SCALES[l]
                    off2 = jnp.int32(l * _TABLE_SIZE * 2)
                    f0, c0, _ = corners_1d(x * sc)
                    f1, c1, _ = corners_1d(y * sc)
                    f2, c2, _ = corners_1d(z * sc)
                    bc = c1 * jnp.uint32(_P1)
                    bf = f1 * jnp.uint32(_P1)
                    dc = c2 * jnp.uint32(_P2)
                    df = f2 * jnp.uint32(_P2)
                    t_cc = c0 ^ bc
                    t_cf = c0 ^ bf
                    t_fc = f0 ^ bc
                    t_ff = f0 ^ bf
                    hs = (
                        t_cc ^ dc,
                        t_cf ^ dc,
                        t_ff ^ dc,
                        t_fc ^ dc,
                        t_cc ^ df,
                        t_cf ^ df,
                        t_ff ^ df,
                        t_fc ^ df,
                    )
                    for k in range(8):
                        e = (hs[k] & jnp.uint32(_MASK)).astype(jnp.int32) * 2 + off2
                        idx_v[pl.ds((l * 8 + k) * 2 * _C + gb, 16)] = e
                        idx_v[pl.ds(((l * 8 + k) * 2 + 1) * _C + gb, 16)] = e + 1
                return carry2

            lax.fori_loop(0, _C // 16, hash_group, 0)

            pltpu.async_copy(table_hbm.at[idx_v], rows_v, gsem).wait()

            def interp_group(g, carry2):
                gb = g * 16
                x = p_v[0, pl.ds(gb, 16)]
                y = p_v[1, pl.ds(gb, 16)]
                z = p_v[2, pl.ds(gb, 16)]
                for l in range(_NUM_LEVELS):
                    sc = _SCALES[l]
                    _, _, o0 = corners_1d(x * sc)
                    _, _, o1 = corners_1d(y * sc)
                    _, _, o2 = corners_1d(z * sc)
                    for f in range(_FEAT):
                        g_ = [
                            rows_v[pl.ds(((l * 8 + k) * 2 + f) * _C + gb, 16)]
                            for k in range(8)
                        ]
                        f03 = g_[3] + o0 * (g_[0] - g_[3])
                        f12 = g_[2] + o0 * (g_[1] - g_[2])
                        f56 = g_[6] + o0 * (g_[5] - g_[6])
                        f47 = g_[7] + o0 * (g_[4] - g_[7])
                        f0312 = f12 + o1 * (f03 - f12)
                        f4756 = f56 + o1 * (f47 - f56)
                        enc_v[l * _FEAT + f, pl.ds(gb, 16)] = (
                            f4756 + o2 * (f0312 - f4756)
                        )
                return carry2

            lax.fori_loop(0, _C // 16, interp_group, 0)
            pltpu.sync_copy(enc_v, enc_hbm.at[:, pl.ds(base, _C)])
            return carry

        lax.fori_loop(0, _CHUNKS, chunk_body, 0)

    return enc_kernel(pT, table_flat)


def _mlp(encT, W1, W2, W3p):
    nb = 512
    grid = _N // nb

    def body(x_ref, w1_ref, w2_ref, w3_ref, o_ref):
        x = x_ref[...]
        h = jnp.maximum(jnp.dot(w1_ref[...], x, preferred_element_type=jnp.float32), 0.0)
        h = jnp.maximum(jnp.dot(w2_ref[...], h, preferred_element_type=jnp.float32), 0.0)
        o_ref[...] = jnp.dot(w3_ref[...], h, preferred_element_type=jnp.float32)

    return pl.pallas_call(
        body,
        grid=(grid,),
        in_specs=[
            pl.BlockSpec((32, nb), lambda i: (0, i)),
            pl.BlockSpec((32, 32), lambda i: (0, 0)),
            pl.BlockSpec((32, 32), lambda i: (0, 0)),
            pl.BlockSpec((8, 32), lambda i: (0, 0)),
        ],
        out_specs=pl.BlockSpec((8, nb), lambda i: (0, i)),
        out_shape=jax.ShapeDtypeStruct((8, _N), jnp.float32),
    )(encT, W1, W2, W3p)


def kernel(p, hash_table, W1, W2, W3):
    pT = p.T
    encT = _sc_encode(pT, hash_table.reshape(-1))
    W3p = jnp.zeros((8, 32), W3.dtype).at[:4, :].set(W3)
    outT = _mlp(encT, W1, W2, W3p)
    return outT[:4, :].T


# speedup vs baseline: 13.1417x; 13.1417x over previous
import functools

import jax
import jax.numpy as jnp
import numpy as np
from jax import lax
from jax.experimental import pallas as pl
from jax.experimental.pallas import tpu as pltpu
from jax.experimental.pallas import tpu_sc as plsc

_NUM_LEVELS = 16
_MIN_RES = 16
_MAX_RES = 1024
_TABLE_SIZE = 2 ** 19
_FEAT = 2
_N = 262144

_growth = np.exp((np.log(_MAX_RES) - np.log(_MIN_RES)) / (_NUM_LEVELS - 1))
_SCALES = [
    float(v)
    for v in np.floor(_MIN_RES * _growth ** np.arange(_NUM_LEVELS)).astype(np.float32)
]
_P1 = 2654435761
_P2 = 805459861
_MASK = _TABLE_SIZE - 1

_LPC = _NUM_LEVELS // 2
_TPTS = _N // 16
_C = 512
_NCH = _TPTS // _C
_CI = _C * 16
_LWORDS = _TABLE_SIZE * 2
_SWORDS = _LWORDS // 16
_EB = 8
_EW = _EB * _C
_PCW = 3 * _C


def _sc_encode(p_flat, table_flat):
    mesh = plsc.VectorSubcoreMesh(core_axis_name="c", subcore_axis_name="s")

    @functools.partial(
        pl.kernel,
        mesh=mesh,
        out_type=jax.ShapeDtypeStruct((_NUM_LEVELS * _FEAT, _N), jnp.float32),
        scratch_types=[
            pltpu.VMEM_SHARED((_LWORDS,), jnp.float32),
            pltpu.VMEM((3 * _PCW,), jnp.float32),
            pltpu.VMEM((2 * _CI,), jnp.int32),
            pltpu.VMEM((2 * _CI,), jnp.float32),
            pltpu.VMEM((_FEAT, _EW), jnp.float32),
            pltpu.SemaphoreType.DMA,
            pltpu.SemaphoreType.DMA,
            pltpu.SemaphoreType.DMA,
            pltpu.SemaphoreType.DMA,
            pltpu.SemaphoreType.DMA,
        ],
    )
    def enc_kernel(p_hbm, table_hbm, enc_hbm, sp, p_v, idx_v, rows_v, enc_v,
                   gsem0, gsem1, psem0, psem1, psem2):
        cid = lax.axis_index("c")
        sid = lax.axis_index("s")
        tbase = sid * _TPTS

        def p_copy(cc, slot):
            return pltpu.make_async_copy(
                p_hbm.at[pl.ds((tbase + cc * _C) * 3, _PCW)],
                p_v.at[pl.ds(slot * _PCW, _PCW)],
                (psem0, psem1, psem2)[slot],
            )

        def fire_p(cc):
            m = lax.rem(cc, 3)
            for s in range(3):
                @pl.when(m == s)
                def _():
                    p_copy(cc, s).start()

        def drain_p(cc):
            m = lax.rem(cc, 3)
            for s in range(3):
                @pl.when(m == s)
                def _():
                    p_copy(cc, s).wait()

        def corners_1d(s):
            fi = s.astype(jnp.int32)
            ff = fi.astype(jnp.float32)
            ci = jnp.where(s == ff, fi, fi + 1)
            return fi.astype(jnp.uint32), ci.astype(jnp.uint32), s - ff

        def coords(po, g):
            q0 = g * 16
            xo = po + ((q0 >> 7) * 384) + (q0 & 127)
            x = p_v[pl.ds(xo, 16)]
            y = p_v[pl.ds(xo + 128, 16)]
            z = p_v[pl.ds(xo + 256, 16)]
            return x, y, z

        def hash_chunk(cc, boff, sv):
            po = lax.rem(cc, 3) * _PCW

            def group(g, carry):
                x, y, z = coords(po, g)
                f0, c0, _ = corners_1d(x * sv)
                f1, c1, _ = corners_1d(y * sv)
                f2, c2, _ = corners_1d(z * sv)
                bc = c1 * jnp.uint32(_P1)
                bf = f1 * jnp.uint32(_P1)
                dc = c2 * jnp.uint32(_P2)
                df = f2 * jnp.uint32(_P2)
                t_cc = c0 ^ bc
                t_cf = c0 ^ bf
                t_fc = f0 ^ bc
                t_ff = f0 ^ bf
                hs = (
                    t_cc ^ dc,
                    t_cf ^ dc,
                    t_ff ^ dc,
                    t_fc ^ dc,
                    t_cc ^ df,
                    t_cf ^ df,
                    t_ff ^ df,
                    t_fc ^ df,
                )
                gb = g * 16
                for k in range(8):
                    v = (hs[k] & jnp.uint32(_MASK)).astype(jnp.int32)
                    e = ((v >> 7) << 8) + (v & 127)
                    idx_v[pl.ds(boff + (k * 2) * _C + gb, 16)] = e
                    idx_v[pl.ds(boff + (k * 2 + 1) * _C + gb, 16)] = e + 128
                return carry

            lax.fori_loop(0, _C // 16, group, 0)

        def fire(boff, parity):
            src = sp.at[idx_v.at[pl.ds(boff, _CI)]]
            dst = rows_v.at[pl.ds(boff, _CI)]

            @pl.when(parity == 0)
            def _():
                pltpu.async_copy(src, dst, gsem0)

            @pl.when(parity != 0)
            def _():
                pltpu.async_copy(src, dst, gsem1)

        def drain(boff, parity):
            src = sp.at[idx_v.at[pl.ds(boff, _CI)]]
            dst = rows_v.at[pl.ds(boff, _CI)]

            @pl.when(parity == 0)
            def _():
                pltpu.make_async_copy(src, dst, gsem0).wait()

            @pl.when(parity != 0)
            def _():
                pltpu.make_async_copy(src, dst, gsem1).wait()

        def interp_chunk(cc, boff, sv):
            ecol = (cc % _EB) * _C
            po = lax.rem(cc, 3) * _PCW

            def group(g, carry):
                x, y, z = coords(po, g)
                _, _, o0 = corners_1d(x * sv)
                _, _, o1 = corners_1d(y * sv)
                _, _, o2 = corners_1d(z * sv)
                gb = g * 16
                for f in range(_FEAT):
                    g_ = [
                        rows_v[pl.ds(boff + (k * 2 + f) * _C + gb, 16)]
                        for k in range(8)
                    ]
                    f03 = g_[3] + o0 * (g_[0] - g_[3])
                    f12 = g_[2] + o0 * (g_[1] - g_[2])
                    f56 = g_[6] + o0 * (g_[5] - g_[6])
                    f47 = g_[7] + o0 * (g_[4] - g_[7])
                    f0312 = f12 + o1 * (f03 - f12)
                    f4756 = f56 + o1 * (f47 - f56)
                    enc_v[f, pl.ds(ecol + gb, 16)] = (
                        f4756 + o2 * (f0312 - f4756)
                    )
                return carry

            lax.fori_loop(0, _C // 16, group, 0)

        def level_pass(lv, carry):
            lg = cid * _LPC + lv
            fire_p(0)
            plsc.subcore_barrier()
            pltpu.sync_copy(
                table_hbm.at[pl.ds(lg * _LWORDS + sid * _SWORDS, _SWORDS)],
                sp.at[pl.ds(sid * _SWORDS, _SWORDS)],
            )
            plsc.subcore_barrier()

            lgv = jnp.full((16,), lg, jnp.int32)
            sv = jnp.full((16,), 0.0, jnp.float32)
            for l in range(_NUM_LEVELS):
                sv = jnp.where(lgv == l, jnp.float32(_SCALES[l]), sv)

            def stage(cc, carry2):
                b = cc & 1
                boff = b * _CI

                @pl.when(cc < _NCH)
                def _():
                    drain_p(cc)
                    hash_chunk(cc, boff, sv)
                    fire(boff, b)

                @pl.when(cc + 1 < _NCH)
                def _():
                    fire_p(cc + 1)

                @pl.when(cc > 0)
                def _():
                    pb = (cc - 1) & 1
                    drain(pb * _CI, pb)
                    interp_chunk(cc - 1, pb * _CI, sv)

                @pl.when(jnp.logical_and(cc > 0, cc % _EB == 0))
                def _():
                    pltpu.sync_copy(
                        enc_v,
                        enc_hbm.at[
                            pl.ds(lg * _FEAT, _FEAT),
                            pl.ds(tbase + (cc // _EB - 1) * _EW, _EW),
                        ],
                    )

                return carry2

            lax.fori_loop(0, _NCH + 1, stage, 0)
            return carry

        lax.fori_loop(0, _LPC, level_pass, 0)

    return enc_kernel(p_flat, table_flat)


def _mlp(encT, W1, W2, W3p):
    nb = 512
    grid = _N // nb

    def body(x_ref, w1_ref, w2_ref, w3_ref, o_ref):
        x = x_ref[...]
        h = jnp.maximum(jnp.dot(w1_ref[...], x, preferred_element_type=jnp.float32), 0.0)
        h = jnp.maximum(jnp.dot(w2_ref[...], h, preferred_element_type=jnp.float32), 0.0)
        o_ref[...] = jnp.dot(w3_ref[...], h, preferred_element_type=jnp.float32)

    return pl.pallas_call(
        body,
        grid=(grid,),
        in_specs=[
            pl.BlockSpec((32, nb), lambda i: (0, i)),
            pl.BlockSpec((32, 32), lambda i: (0, 0)),
            pl.BlockSpec((32, 32), lambda i: (0, 0)),
            pl.BlockSpec((8, 32), lambda i: (0, 0)),
        ],
        out_specs=pl.BlockSpec((8, nb), lambda i: (0, i)),
        out_shape=jax.ShapeDtypeStruct((8, _N), jnp.float32),
    )(encT, W1, W2, W3p)


def kernel(p, hash_table, W1, W2, W3):
    p_flat = p.reshape(2048, 128, 3).transpose(0, 2, 1).reshape(-1)
    table_flat = hash_table.reshape(65536, 128, 2).transpose(0, 2, 1).reshape(-1)
    encT = _sc_encode(p_flat, table_flat)
    W3p = jnp.zeros((8, 32), W3.dtype).at[:4, :].set(W3)
    outT = _mlp(encT, W1, W2, W3p)
    return outT[:4, :].T


# speedup vs baseline: 16.8242x; 1.2802x over previous
import functools

import jax
import jax.numpy as jnp
import numpy as np
from jax import lax
from jax.experimental import pallas as pl
from jax.experimental.pallas import tpu as pltpu
from jax.experimental.pallas import tpu_sc as plsc

_NUM_LEVELS = 16
_MIN_RES = 16
_MAX_RES = 1024
_TABLE_SIZE = 2 ** 19
_FEAT = 2
_N = 262144

_growth = np.exp((np.log(_MAX_RES) - np.log(_MIN_RES)) / (_NUM_LEVELS - 1))
_SCALES = [
    float(v)
    for v in np.floor(_MIN_RES * _growth ** np.arange(_NUM_LEVELS)).astype(np.float32)
]
_P1 = 2654435761
_P2 = 805459861
_MASK = _TABLE_SIZE - 1

_LPC = _NUM_LEVELS // 2
_TPTS = _N // 16
_C = 512
_NCH = _TPTS // _C
_CI = _C * 16
_LWORDS = _TABLE_SIZE * 2
_SWORDS = _LWORDS // 16
_EB = 8
_EW = _EB * _C
_PCW = 3 * _C


def _sc_encode(p_flat, table_flat):
    mesh = plsc.VectorSubcoreMesh(core_axis_name="c", subcore_axis_name="s")

    @functools.partial(
        pl.kernel,
        mesh=mesh,
        out_type=jax.ShapeDtypeStruct((_NUM_LEVELS * _FEAT, _N), jnp.float32),
        scratch_types=[
            pltpu.VMEM_SHARED((_LWORDS,), jnp.float32),
            pltpu.VMEM((3 * _PCW,), jnp.float32),
            pltpu.VMEM((2 * _CI,), jnp.int32),
            pltpu.VMEM((2 * _CI,), jnp.float32),
            pltpu.VMEM((_FEAT, _EW), jnp.float32),
            pltpu.SemaphoreType.DMA,
            pltpu.SemaphoreType.DMA,
            pltpu.SemaphoreType.DMA,
            pltpu.SemaphoreType.DMA,
            pltpu.SemaphoreType.DMA,
        ],
    )
    def enc_kernel(p_hbm, table_hbm, enc_hbm, sp, p_v, idx_v, rows_v, enc_v,
                   gsem0, gsem1, psem0, psem1, psem2):
        cid = lax.axis_index("c")
        sid = lax.axis_index("s")
        tbase = sid * _TPTS

        def p_copy(cc, slot):
            return pltpu.make_async_copy(
                p_hbm.at[pl.ds((tbase + cc * _C) * 3, _PCW)],
                p_v.at[pl.ds(slot * _PCW, _PCW)],
                (psem0, psem1, psem2)[slot],
            )

        def fire_p(cc):
            m = lax.rem(cc, 3)
            for s in range(3):
                @pl.when(m == s)
                def _():
                    p_copy(cc, s).start()

        def drain_p(cc):
            m = lax.rem(cc, 3)
            for s in range(3):
                @pl.when(m == s)
                def _():
                    p_copy(cc, s).wait()

        def corners_1d(s):
            fi = s.astype(jnp.int32)
            ff = fi.astype(jnp.float32)
            ci = jnp.where(s == ff, fi, fi + 1)
            return fi.astype(jnp.uint32), ci.astype(jnp.uint32), s - ff

        def coords(po, g):
            q0 = g * 16
            xo = po + ((q0 >> 7) * 384) + (q0 & 127)
            x = p_v[pl.ds(xo, 16)]
            y = p_v[pl.ds(xo + 128, 16)]
            z = p_v[pl.ds(xo + 256, 16)]
            return x, y, z

        def hash_chunk(cc, boff, sv):
            po = lax.rem(cc, 3) * _PCW

            def group(g, carry):
                x, y, z = coords(po, g)
                f0, c0, _ = corners_1d(x * sv)
                f1, c1, _ = corners_1d(y * sv)
                f2, c2, _ = corners_1d(z * sv)
                bc = c1 * jnp.uint32(_P1)
                bf = f1 * jnp.uint32(_P1)
                dc = c2 * jnp.uint32(_P2)
                df = f2 * jnp.uint32(_P2)
                t_cc = c0 ^ bc
                t_cf = c0 ^ bf
                t_fc = f0 ^ bc
                t_ff = f0 ^ bf
                hs = (
                    t_cc ^ dc,
                    t_cf ^ dc,
                    t_ff ^ dc,
                    t_fc ^ dc,
                    t_cc ^ df,
                    t_cf ^ df,
                    t_ff ^ df,
                    t_fc ^ df,
                )
                gb = g * 16
                for k in range(8):
                    v = (hs[k] & jnp.uint32(_MASK)).astype(jnp.int32)
                    e = ((v >> 7) << 8) + (v & 127)
                    idx_v[pl.ds(boff + (k * 2) * _C + gb, 16)] = e
                    idx_v[pl.ds(boff + (k * 2 + 1) * _C + gb, 16)] = e + 128
                return carry

            lax.fori_loop(0, _C // 16, group, 0)

        def fire(boff, parity):
            src = sp.at[idx_v.at[pl.ds(boff, _CI)]]
            dst = rows_v.at[pl.ds(boff, _CI)]

            @pl.when(parity == 0)
            def _():
                pltpu.async_copy(src, dst, gsem0)

            @pl.when(parity != 0)
            def _():
                pltpu.async_copy(src, dst, gsem1)

        def drain(boff, parity):
            src = sp.at[idx_v.at[pl.ds(boff, _CI)]]
            dst = rows_v.at[pl.ds(boff, _CI)]

            @pl.when(parity == 0)
            def _():
                pltpu.make_async_copy(src, dst, gsem0).wait()

            @pl.when(parity != 0)
            def _():
                pltpu.make_async_copy(src, dst, gsem1).wait()

        def interp_chunk(cc, boff, sv):
            ecol = (cc % _EB) * _C
            po = lax.rem(cc, 3) * _PCW

            def group(g, carry):
                x, y, z = coords(po, g)
                _, _, o0 = corners_1d(x * sv)
                _, _, o1 = corners_1d(y * sv)
                _, _, o2 = corners_1d(z * sv)
                gb = g * 16
                for f in range(_FEAT):
                    g_ = [
                        rows_v[pl.ds(boff + (k * 2 + f) * _C + gb, 16)]
                        for k in range(8)
                    ]
                    f03 = g_[3] + o0 * (g_[0] - g_[3])
                    f12 = g_[2] + o0 * (g_[1] - g_[2])
                    f56 = g_[6] + o0 * (g_[5] - g_[6])
                    f47 = g_[7] + o0 * (g_[4] - g_[7])
                    f0312 = f12 + o1 * (f03 - f12)
                    f4756 = f56 + o1 * (f47 - f56)
                    enc_v[f, pl.ds(ecol + gb, 16)] = (
                        f4756 + o2 * (f0312 - f4756)
                    )
                return carry

            lax.fori_loop(0, _C // 16, group, 0)

        def level_pass(lv, carry):
            lg = cid * _LPC + lv
            fire_p(0)
            plsc.subcore_barrier()
            pltpu.sync_copy(
                table_hbm.at[pl.ds(lg * _LWORDS + sid * _SWORDS, _SWORDS)],
                sp.at[pl.ds(sid * _SWORDS, _SWORDS)],
            )
            plsc.subcore_barrier()

            lgv = jnp.full((16,), lg, jnp.int32)
            sv = jnp.full((16,), 0.0, jnp.float32)
            for l in range(_NUM_LEVELS):
                sv = jnp.where(lgv == l, jnp.float32(_SCALES[l]), sv)

            def stage(cc, carry2):
                b = cc & 1
                boff = b * _CI

                @pl.when(cc < _NCH)
                def _():
                    drain_p(cc)
                    hash_chunk(cc, boff, sv)
                    fire(boff, b)

                @pl.when(cc + 1 < _NCH)
                def _():
                    fire_p(cc + 1)

                @pl.when(cc > 0)
                def _():
                    pb = (cc - 1) & 1
                    drain(pb * _CI, pb)
                    interp_chunk(cc - 1, pb * _CI, sv)

                @pl.when(jnp.logical_and(cc > 0, cc % _EB == 0))
                def _():
                    pltpu.sync_copy(
                        enc_v,
                        enc_hbm.at[
                            pl.ds(lg * _FEAT, _FEAT),
                            pl.ds(tbase + (cc // _EB - 1) * _EW, _EW),
                        ],
                    )

                return carry2

            lax.fori_loop(0, _NCH + 1, stage, 0)
            return carry

        lax.fori_loop(0, _LPC, level_pass, 0)

    return enc_kernel(p_flat, table_flat)


def _mlp(encT, W1, W2, W3p):
    nb = 2048
    grid = _N // nb

    def body(x_ref, w1_ref, w2_ref, w3_ref, o_ref):
        x = x_ref[...]
        h = jnp.maximum(jnp.dot(w1_ref[...], x, preferred_element_type=jnp.float32), 0.0)
        h = jnp.maximum(jnp.dot(w2_ref[...], h, preferred_element_type=jnp.float32), 0.0)
        o_ref[...] = jnp.dot(w3_ref[...], h, preferred_element_type=jnp.float32)

    return pl.pallas_call(
        body,
        grid=(grid,),
        in_specs=[
            pl.BlockSpec((32, nb), lambda i: (0, i)),
            pl.BlockSpec((32, 32), lambda i: (0, 0)),
            pl.BlockSpec((32, 32), lambda i: (0, 0)),
            pl.BlockSpec((8, 32), lambda i: (0, 0)),
        ],
        out_specs=pl.BlockSpec((8, nb), lambda i: (0, i)),
        out_shape=jax.ShapeDtypeStruct((8, _N), jnp.float32),
    )(encT, W1, W2, W3p)


def kernel(p, hash_table, W1, W2, W3):
    p_flat = p.reshape(2048, 128, 3).transpose(0, 2, 1).reshape(-1)
    table_flat = hash_table.reshape(65536, 128, 2).transpose(0, 2, 1).reshape(-1)
    encT = _sc_encode(p_flat, table_flat)
    W3p = jnp.zeros((8, 32), W3.dtype).at[:4, :].set(W3)
    outT = _mlp(encT, W1, W2, W3p)
    return outT[:4, :].T


# speedup vs baseline: 16.8403x; 1.0010x over previous
import functools

import jax
import jax.numpy as jnp
import numpy as np
from jax import lax
from jax.experimental import pallas as pl
from jax.experimental.pallas import tpu as pltpu
from jax.experimental.pallas import tpu_sc as plsc

_NUM_LEVELS = 16
_MIN_RES = 16
_MAX_RES = 1024
_TABLE_SIZE = 2 ** 19
_FEAT = 2
_N = 262144

_growth = np.exp((np.log(_MAX_RES) - np.log(_MIN_RES)) / (_NUM_LEVELS - 1))
_SCALES = [
    float(v)
    for v in np.floor(_MIN_RES * _growth ** np.arange(_NUM_LEVELS)).astype(np.float32)
]
_P1 = 2654435761
_P2 = 805459861
_MASK = _TABLE_SIZE - 1

_LPC = _NUM_LEVELS // 2
_TPTS = _N // 16
_C = 512
_NCH = _TPTS // _C
_CI = _C * 16
_LWORDS = _TABLE_SIZE * 2
_SWORDS = _LWORDS // 16
_EB = 8
_EW = _EB * _C
_PCW = 3 * _C


def _sc_encode(p_flat, table_flat):
    mesh = plsc.VectorSubcoreMesh(core_axis_name="c", subcore_axis_name="s")

    @functools.partial(
        pl.kernel,
        mesh=mesh,
        out_type=jax.ShapeDtypeStruct((_NUM_LEVELS * _FEAT, _N), jnp.float32),
        scratch_types=[
            pltpu.VMEM_SHARED((_LWORDS,), jnp.float32),
            pltpu.VMEM((3 * _PCW,), jnp.float32),
            pltpu.VMEM((2 * _CI,), jnp.int32),
            pltpu.VMEM((2 * _CI,), jnp.float32),
            pltpu.VMEM((_FEAT, _EW), jnp.float32),
            pltpu.VMEM((2 * 3 * _C,), jnp.float32),
            pltpu.SemaphoreType.DMA,
            pltpu.SemaphoreType.DMA,
            pltpu.SemaphoreType.DMA,
            pltpu.SemaphoreType.DMA,
            pltpu.SemaphoreType.DMA,
        ],
    )
    def enc_kernel(p_hbm, table_hbm, enc_hbm, sp, p_v, idx_v, rows_v, enc_v,
                   off_v, gsem0, gsem1, psem0, psem1, psem2):
        cid = lax.axis_index("c")
        sid = lax.axis_index("s")
        tbase = sid * _TPTS

        def p_copy(cc, slot):
            return pltpu.make_async_copy(
                p_hbm.at[pl.ds((tbase + cc * _C) * 3, _PCW)],
                p_v.at[pl.ds(slot * _PCW, _PCW)],
                (psem0, psem1, psem2)[slot],
            )

        def fire_p(cc):
            m = lax.rem(cc, 3)
            for s in range(3):
                @pl.when(m == s)
                def _():
                    p_copy(cc, s).start()

        def drain_p(cc):
            m = lax.rem(cc, 3)
            for s in range(3):
                @pl.when(m == s)
                def _():
                    p_copy(cc, s).wait()

        def corners_1d(s):
            fi = s.astype(jnp.int32)
            ff = fi.astype(jnp.float32)
            ci = jnp.where(s == ff, fi, fi + 1)
            return fi.astype(jnp.uint32), ci.astype(jnp.uint32), s - ff

        def coords(po, g):
            q0 = g * 16
            xo = po + ((q0 >> 7) * 384) + (q0 & 127)
            x = p_v[pl.ds(xo, 16)]
            y = p_v[pl.ds(xo + 128, 16)]
            z = p_v[pl.ds(xo + 256, 16)]
            return x, y, z

        def hash_chunk(cc, boff, sv):
            po = lax.rem(cc, 3) * _PCW

            def group(g, carry):
                x, y, z = coords(po, g)
                f0, c0, o0 = corners_1d(x * sv)
                f1, c1, o1 = corners_1d(y * sv)
                f2, c2, o2 = corners_1d(z * sv)
                gb = g * 16
                woff = (boff // _CI) * 3 * _C
                off_v[pl.ds(woff + gb, 16)] = o0
                off_v[pl.ds(woff + _C + gb, 16)] = o1
                off_v[pl.ds(woff + 2 * _C + gb, 16)] = o2
                bc = c1 * jnp.uint32(_P1)
                bf = f1 * jnp.uint32(_P1)
                dc = c2 * jnp.uint32(_P2)
                df = f2 * jnp.uint32(_P2)
                t_cc = c0 ^ bc
                t_cf = c0 ^ bf
                t_fc = f0 ^ bc
                t_ff = f0 ^ bf
                hs = (
                    t_cc ^ dc,
                    t_cf ^ dc,
                    t_ff ^ dc,
                    t_fc ^ dc,
                    t_cc ^ df,
                    t_cf ^ df,
                    t_ff ^ df,
                    t_fc ^ df,
                )
                for k in range(8):
                    v = hs[k] & jnp.uint32(_MASK)
                    e = (v + (v & jnp.uint32(0xFFFFFF80))).astype(jnp.int32)
                    idx_v[pl.ds(boff + (k * 2) * _C + gb, 16)] = e
                    idx_v[pl.ds(boff + (k * 2 + 1) * _C + gb, 16)] = e + 128
                return carry

            lax.fori_loop(0, _C // 16, group, 0)

        def fire(boff, parity):
            src = sp.at[idx_v.at[pl.ds(boff, _CI)]]
            dst = rows_v.at[pl.ds(boff, _CI)]

            @pl.when(parity == 0)
            def _():
                pltpu.async_copy(src, dst, gsem0)

            @pl.when(parity != 0)
            def _():
                pltpu.async_copy(src, dst, gsem1)

        def drain(boff, parity):
            src = sp.at[idx_v.at[pl.ds(boff, _CI)]]
            dst = rows_v.at[pl.ds(boff, _CI)]

            @pl.when(parity == 0)
            def _():
                pltpu.make_async_copy(src, dst, gsem0).wait()

            @pl.when(parity != 0)
            def _():
                pltpu.make_async_copy(src, dst, gsem1).wait()

        def interp_chunk(cc, boff, sv):
            ecol = (cc % _EB) * _C

            def group(g, carry):
                gb = g * 16
                woff = (boff // _CI) * 3 * _C
                o0 = off_v[pl.ds(woff + gb, 16)]
                o1 = off_v[pl.ds(woff + _C + gb, 16)]
                o2 = off_v[pl.ds(woff + 2 * _C + gb, 16)]
                for f in range(_FEAT):
                    g_ = [
                        rows_v[pl.ds(boff + (k * 2 + f) * _C + gb, 16)]
                        for k in range(8)
                    ]
                    f03 = g_[3] + o0 * (g_[0] - g_[3])
                    f12 = g_[2] + o0 * (g_[1] - g_[2])
                    f56 = g_[6] + o0 * (g_[5] - g_[6])
                    f47 = g_[7] + o0 * (g_[4] - g_[7])
                    f0312 = f12 + o1 * (f03 - f12)
                    f4756 = f56 + o1 * (f47 - f56)
                    enc_v[f, pl.ds(ecol + gb, 16)] = (
                        f4756 + o2 * (f0312 - f4756)
                    )
                return carry

            lax.fori_loop(0, _C // 16, group, 0)

        def level_pass(lv, carry):
            lg = cid * _LPC + lv
            fire_p(0)
            plsc.subcore_barrier()
            pltpu.sync_copy(
                table_hbm.at[pl.ds(lg * _LWORDS + sid * _SWORDS, _SWORDS)],
                sp.at[pl.ds(sid * _SWORDS, _SWORDS)],
            )
            plsc.subcore_barrier()

            lgv = jnp.full((16,), lg, jnp.int32)
            sv = jnp.full((16,), 0.0, jnp.float32)
            for l in range(_NUM_LEVELS):
                sv = jnp.where(lgv == l, jnp.float32(_SCALES[l]), sv)

            def stage(cc, carry2):
                b = cc & 1
                boff = b * _CI

                @pl.when(cc < _NCH)
                def _():
                    drain_p(cc)
                    hash_chunk(cc, boff, sv)
                    fire(boff, b)

                @pl.when(cc + 1 < _NCH)
                def _():
                    fire_p(cc + 1)

                @pl.when(cc > 0)
                def _():
                    pb = (cc - 1) & 1
                    drain(pb * _CI, pb)
                    interp_chunk(cc - 1, pb * _CI, sv)

                @pl.when(jnp.logical_and(cc > 0, cc % _EB == 0))
                def _():
                    pltpu.sync_copy(
                        enc_v,
                        enc_hbm.at[
                            pl.ds(lg * _FEAT, _FEAT),
                            pl.ds(tbase + (cc // _EB - 1) * _EW, _EW),
                        ],
                    )

                return carry2

            lax.fori_loop(0, _NCH + 1, stage, 0)
            return carry

        lax.fori_loop(0, _LPC, level_pass, 0)

    return enc_kernel(p_flat, table_flat)


def _mlp(encT, W1, W2, W3p):
    nb = 2048
    grid = _N // nb

    def body(x_ref, w1_ref, w2_ref, w3_ref, o_ref):
        x = x_ref[...]
        h = jnp.maximum(jnp.dot(w1_ref[...], x, preferred_element_type=jnp.float32), 0.0)
        h = jnp.maximum(jnp.dot(w2_ref[...], h, preferred_element_type=jnp.float32), 0.0)
        o_ref[...] = jnp.dot(w3_ref[...], h, preferred_element_type=jnp.float32)

    return pl.pallas_call(
        body,
        grid=(grid,),
        in_specs=[
            pl.BlockSpec((32, nb), lambda i: (0, i)),
            pl.BlockSpec((32, 32), lambda i: (0, 0)),
            pl.BlockSpec((32, 32), lambda i: (0, 0)),
            pl.BlockSpec((8, 32), lambda i: (0, 0)),
        ],
        out_specs=pl.BlockSpec((8, nb), lambda i: (0, i)),
        out_shape=jax.ShapeDtypeStruct((8, _N), jnp.float32),
    )(encT, W1, W2, W3p)


def kernel(p, hash_table, W1, W2, W3):
    p_flat = p.reshape(2048, 128, 3).transpose(0, 2, 1).reshape(-1)
    table_flat = hash_table.reshape(65536, 128, 2).transpose(0, 2, 1).reshape(-1)
    encT = _sc_encode(p_flat, table_flat)
    W3p = jnp.zeros((8, 32), W3.dtype).at[:4, :].set(W3)
    outT = _mlp(encT, W1, W2, W3p)
    return outT[:4, :].T
